# Initial kernel scaffold; baseline (speedup 1.0000x reference)
import jax
import jax.numpy as jnp
from jax.experimental import pallas as pl


def kernel(x, edge_index, W_msg, b_msg, W_lin, b_lin):
    raise NotImplementedError("write your pallas kernel here")


# speedup vs baseline: 11.4590x; 11.4590x over previous
import dataclasses
import functools

import jax
import jax.numpy as jnp
from jax import lax
from jax.experimental import pallas as pl
from jax.experimental.pallas import tpu as pltpu
from jax.experimental.pallas import tpu_sc as plsc

NUM_CORES = 2
NUM_SUBCORES = 16
NUM_WORKERS = NUM_CORES * NUM_SUBCORES
CHUNK = 128

_SC_PARAMS = pltpu.CompilerParams()
if "needs_layout_passes" in pltpu.CompilerParams.__dataclass_fields__:
    _SC_PARAMS = dataclasses.replace(_SC_PARAMS, needs_layout_passes=False)


def _sc_degree(dst2d, np_pad):
    rows, chunk = dst2d.shape
    rows_per_w = rows // NUM_WORKERS
    npr = np_pad // 128
    mesh = plsc.VectorSubcoreMesh(core_axis_name="c", subcore_axis_name="s")

    @functools.partial(
        pl.kernel,
        out_type=jax.ShapeDtypeStruct((NUM_WORKERS, npr, 128), jnp.float32),
        mesh=mesh,
        compiler_params=_SC_PARAMS,
        scratch_types=[
            pltpu.VMEM((rows_per_w, chunk), jnp.int32),
            pltpu.VMEM((npr, 128), jnp.float32),
        ],
    )
    def deg_kernel(dst_hbm, deg_out, didx_v, deg_v):
        cid = lax.axis_index("c")
        sid = lax.axis_index("s")
        wid = cid * NUM_SUBCORES + sid

        @pl.loop(0, npr)
        def _(i):
            @pl.loop(0, 8)
            def _(kk):
                deg_v[i, pl.ds(kk * 16, 16)] = jnp.zeros((16,), jnp.float32)

        pltpu.sync_copy(dst_hbm.at[pl.ds(wid * rows_per_w, rows_per_w)], didx_v)

        ones16 = jnp.ones((16,), jnp.float32)

        @pl.loop(0, rows_per_w)
        def _(j):
            @pl.loop(0, chunk // 16)
            def _(kk):
                idx = didx_v[j, pl.ds(kk * 16, 16)]
                row = lax.shift_right_logical(idx, 7)
                col = lax.bitwise_and(idx, 127)
                plsc.addupdate_scatter(deg_v, [row, col], ones16)

        pltpu.sync_copy(deg_v, deg_out.at[wid])

    return deg_kernel(dst2d)


def _sc_gather_scatter(h2, src2d, dst2d):
    np_pad, d = h2.shape
    rows, chunk = src2d.shape
    rows_per_w = rows // NUM_WORKERS
    stripe = np_pad // NUM_SUBCORES
    zc = stripe // CHUNK
    mesh = plsc.VectorSubcoreMesh(core_axis_name="c", subcore_axis_name="s")

    @functools.partial(
        pl.kernel,
        out_type=jax.ShapeDtypeStruct((NUM_CORES, np_pad, d), jnp.float32),
        mesh=mesh,
        compiler_params=_SC_PARAMS,
        scratch_types=[
            pltpu.VMEM((rows_per_w, chunk), jnp.int32),
            pltpu.VMEM((rows_per_w, chunk), jnp.int32),
            pltpu.VMEM((CHUNK, d), jnp.float32),
            pltpu.VMEM((zc, CHUNK), jnp.int32),
            pltpu.VMEM_SHARED((np_pad, d), jnp.float32),
            pltpu.SemaphoreType.DMA,
        ],
    )
    def gs_kernel(h2_hbm, src_hbm, dst_hbm, agg_out,
                  sidx_v, didx_v, buf_v, zidx_v, agg_sh, sem):
        cid = lax.axis_index("c")
        sid = lax.axis_index("s")
        wid = cid * NUM_SUBCORES + sid

        @pl.loop(0, CHUNK)
        def _(i):
            @pl.loop(0, d // 16)
            def _(kk):
                buf_v[i, pl.ds(kk * 16, 16)] = jnp.zeros((16,), jnp.float32)

        @pl.loop(0, zc)
        def _(c):
            @pl.loop(0, CHUNK // 16)
            def _(kk):
                zidx_v[c, pl.ds(kk * 16, 16)] = (
                    lax.iota(jnp.int32, 16) + (sid * stripe + kk * 16)
                    + c * CHUNK)

        for c in range(zc):
            pltpu.sync_copy(buf_v, agg_sh.at[zidx_v.at[c]])

        pltpu.sync_copy(src_hbm.at[pl.ds(wid * rows_per_w, rows_per_w)], sidx_v)
        pltpu.sync_copy(dst_hbm.at[pl.ds(wid * rows_per_w, rows_per_w)], didx_v)
        plsc.subcore_barrier()

        @pl.loop(0, rows_per_w)
        def _(j):
            pltpu.async_copy(h2_hbm.at[sidx_v.at[j]], buf_v, sem).wait()
            pltpu.sync_copy(buf_v, agg_sh.at[didx_v.at[j]], add=True)

        plsc.subcore_barrier()

        for c in range(zc):
            pltpu.sync_copy(agg_sh.at[zidx_v.at[c]], buf_v)
            pltpu.sync_copy(
                buf_v,
                agg_out.at[cid, pl.ds(sid * stripe + c * CHUNK, CHUNK)])

    return gs_kernel(h2, src2d, dst2d)


def _tc_transform(x_pad, W_msg, b_msg, W_lin, b_lin, deg_t, block_rows):
    np_pad, d = x_pad.shape

    def body(x_ref, wm_ref, bm_ref, wl_ref, bl_ref, deg_ref, h2_ref, skip_ref):
        deg = jnp.sum(deg_ref[...], axis=1, keepdims=True) + 1.0
        isd = lax.rsqrt(deg)
        h = jnp.dot(x_ref[...], wm_ref[...], preferred_element_type=jnp.float32)
        h2_ref[...] = (h + bm_ref[...]) * isd
        skip_ref[...] = (
            jnp.dot(x_ref[...], wl_ref[...], preferred_element_type=jnp.float32)
            + bl_ref[...])

    return pl.pallas_call(
        body,
        grid=(np_pad // block_rows,),
        in_specs=[
            pl.BlockSpec((block_rows, d), lambda i: (i, 0)),
            pl.BlockSpec((d, d), lambda i: (0, 0)),
            pl.BlockSpec((d,), lambda i: (0,)),
            pl.BlockSpec((d, d), lambda i: (0, 0)),
            pl.BlockSpec((d,), lambda i: (0,)),
            pl.BlockSpec((block_rows, NUM_WORKERS), lambda i: (i, 0)),
        ],
        out_specs=[
            pl.BlockSpec((block_rows, d), lambda i: (i, 0)),
            pl.BlockSpec((block_rows, d), lambda i: (i, 0)),
        ],
        out_shape=[
            jax.ShapeDtypeStruct((np_pad, d), jnp.float32),
            jax.ShapeDtypeStruct((np_pad, d), jnp.float32),
        ],
    )(x_pad, W_msg, b_msg, W_lin, b_lin, deg_t)


def _tc_final(agg_parts, deg_t, skip, block_rows):
    _, np_pad, d = agg_parts.shape

    def body(agg_ref, deg_ref, skip_ref, out_ref):
        deg = jnp.sum(deg_ref[...], axis=1, keepdims=True) + 1.0
        isd = lax.rsqrt(deg)
        seg = agg_ref[0] + agg_ref[1]
        out_ref[...] = jnp.maximum(seg * isd + skip_ref[...], 0.0)

    return pl.pallas_call(
        body,
        grid=(np_pad // block_rows,),
        in_specs=[
            pl.BlockSpec((NUM_CORES, block_rows, d), lambda i: (0, i, 0)),
            pl.BlockSpec((block_rows, NUM_WORKERS), lambda i: (i, 0)),
            pl.BlockSpec((block_rows, d), lambda i: (i, 0)),
        ],
        out_specs=pl.BlockSpec((block_rows, d), lambda i: (i, 0)),
        out_shape=jax.ShapeDtypeStruct((np_pad, d), jnp.float32),
    )(agg_parts, deg_t, skip)


def kernel(x, edge_index, W_msg, b_msg, W_lin, b_lin):
    n, d = x.shape
    e = edge_index.shape[1]
    np_pad = ((n + 2047) // 2048) * 2048
    e_unit = CHUNK * NUM_WORKERS * 8
    e_pad = ((e + e_unit - 1) // e_unit) * e_unit

    x_pad = jnp.pad(x, ((0, np_pad - n), (0, 0)))
    ei_pad = jnp.pad(edge_index, ((0, 0), (0, e_pad - e)), constant_values=n)
    src2d = ei_pad[0].reshape(e_pad // CHUNK, CHUNK)
    dst2d = ei_pad[1].reshape(e_pad // CHUNK, CHUNK)

    deg_parts = _sc_degree(dst2d, np_pad)
    deg_t = deg_parts.reshape(NUM_WORKERS, np_pad).T
    h2, skip = _tc_transform(x_pad, W_msg, b_msg, W_lin, b_lin, deg_t, 640)
    agg_parts = _sc_gather_scatter(h2, src2d, dst2d)
    out = _tc_final(agg_parts, deg_t, skip, 640)
    return out[:n]


# speedup vs baseline: 11.8912x; 1.0377x over previous
import dataclasses
import functools

import jax
import jax.numpy as jnp
from jax import lax
from jax.experimental import pallas as pl
from jax.experimental.pallas import tpu as pltpu
from jax.experimental.pallas import tpu_sc as plsc

NUM_CORES = 2
NUM_SUBCORES = 16
NUM_WORKERS = NUM_CORES * NUM_SUBCORES
CHUNK = 128

_SC_PARAMS = pltpu.CompilerParams()
if "needs_layout_passes" in pltpu.CompilerParams.__dataclass_fields__:
    _SC_PARAMS = dataclasses.replace(_SC_PARAMS, needs_layout_passes=False)


def _sc_degree(dst2d, np_pad):
    rows, chunk = dst2d.shape
    rows_per_w = rows // NUM_WORKERS
    npr = np_pad // 128
    mesh = plsc.VectorSubcoreMesh(core_axis_name="c", subcore_axis_name="s")

    @functools.partial(
        pl.kernel,
        out_type=jax.ShapeDtypeStruct((NUM_WORKERS, npr, 128), jnp.float32),
        mesh=mesh,
        compiler_params=_SC_PARAMS,
        scratch_types=[
            pltpu.VMEM((rows_per_w, chunk), jnp.int32),
            pltpu.VMEM((npr, 128), jnp.float32),
        ],
    )
    def deg_kernel(dst_hbm, deg_out, didx_v, deg_v):
        cid = lax.axis_index("c")
        sid = lax.axis_index("s")
        wid = cid * NUM_SUBCORES + sid

        @pl.loop(0, npr)
        def _(i):
            @pl.loop(0, 8)
            def _(kk):
                deg_v[i, pl.ds(kk * 16, 16)] = jnp.zeros((16,), jnp.float32)

        pltpu.sync_copy(dst_hbm.at[pl.ds(wid * rows_per_w, rows_per_w)], didx_v)

        ones16 = jnp.ones((16,), jnp.float32)

        @pl.loop(0, rows_per_w)
        def _(j):
            @pl.loop(0, chunk // 16)
            def _(kk):
                idx = didx_v[j, pl.ds(kk * 16, 16)]
                row = lax.shift_right_logical(idx, 7)
                col = lax.bitwise_and(idx, 127)
                plsc.addupdate_scatter(deg_v, [row, col], ones16)

        pltpu.sync_copy(deg_v, deg_out.at[wid])

    return deg_kernel(dst2d)


def _sc_gather_scatter(h2, src2d, dst2d):
    np_pad, d = h2.shape
    rows, chunk = src2d.shape
    rows_per_w = rows // NUM_WORKERS
    half = rows_per_w // 2
    stripe = np_pad // NUM_SUBCORES
    zc = stripe // CHUNK
    mesh = plsc.VectorSubcoreMesh(core_axis_name="c", subcore_axis_name="s")

    @functools.partial(
        pl.kernel,
        out_type=jax.ShapeDtypeStruct((NUM_CORES, np_pad, d), jnp.float32),
        mesh=mesh,
        compiler_params=_SC_PARAMS,
        scratch_types=[
            pltpu.VMEM((rows_per_w // 2, chunk), jnp.int32),
            pltpu.VMEM((rows_per_w // 2, chunk), jnp.int32),
            pltpu.VMEM((2 * CHUNK, d), jnp.float32),
            pltpu.VMEM((zc, CHUNK), jnp.int32),
            pltpu.VMEM_SHARED((np_pad, d), jnp.float32),
            pltpu.SemaphoreType.DMA,
            pltpu.SemaphoreType.DMA,
        ],
    )
    def gs_kernel(h2_hbm, src_hbm, dst_hbm, agg_out,
                  sidx_v, didx_v, bufab_v, zidx_v, agg_sh, sema, semb):
        buf_v = bufab_v.at[pl.ds(0, CHUNK)]
        bufb_v = bufab_v.at[pl.ds(CHUNK, CHUNK)]
        cid = lax.axis_index("c")
        sid = lax.axis_index("s")
        wid = cid * NUM_SUBCORES + sid

        @pl.loop(0, 2 * CHUNK)
        def _(i):
            @pl.loop(0, d // 16)
            def _(kk):
                bufab_v[i, pl.ds(kk * 16, 16)] = jnp.zeros((16,), jnp.float32)

        @pl.loop(0, zc)
        def _(c):
            @pl.loop(0, CHUNK // 16)
            def _(kk):
                zidx_v[c, pl.ds(kk * 16, 16)] = (
                    lax.iota(jnp.int32, 16) + (sid * stripe + kk * 16)
                    + c * CHUNK)

        for c in range(zc):
            pltpu.sync_copy(buf_v, agg_sh.at[zidx_v.at[c]])

        plsc.subcore_barrier()

        for h in range(2):
            base = wid * rows_per_w + h * half
            pltpu.sync_copy(src_hbm.at[pl.ds(base, half)], sidx_v)
            pltpu.sync_copy(dst_hbm.at[pl.ds(base, half)], didx_v)
            pltpu.async_copy(h2_hbm.at[sidx_v.at[0]], buf_v, sema)

            @pl.loop(0, half // 2)
            def _(g):
                j0 = 2 * g
                j1 = 2 * g + 1
                j2 = jnp.minimum(2 * g + 2, half - 1)
                pltpu.make_async_copy(
                    h2_hbm.at[sidx_v.at[j0]], buf_v, sema).wait()
                pltpu.async_copy(h2_hbm.at[sidx_v.at[j1]], bufb_v, semb)
                pltpu.sync_copy(buf_v, agg_sh.at[didx_v.at[j0]], add=True)
                pltpu.make_async_copy(
                    h2_hbm.at[sidx_v.at[j1]], bufb_v, semb).wait()
                pltpu.async_copy(h2_hbm.at[sidx_v.at[j2]], buf_v, sema)
                pltpu.sync_copy(bufb_v, agg_sh.at[didx_v.at[j1]], add=True)

            pltpu.make_async_copy(
                h2_hbm.at[sidx_v.at[half - 1]], buf_v, sema).wait()

        plsc.subcore_barrier()

        for c in range(zc):
            pltpu.sync_copy(agg_sh.at[zidx_v.at[c]], buf_v)
            pltpu.sync_copy(
                buf_v,
                agg_out.at[cid, pl.ds(sid * stripe + c * CHUNK, CHUNK)])

    return gs_kernel(h2, src2d, dst2d)


def _tc_transform(x_pad, W_msg, b_msg, W_lin, b_lin, deg_t, block_rows):
    np_pad, d = x_pad.shape

    def body(x_ref, wm_ref, bm_ref, wl_ref, bl_ref, deg_ref, h2_ref, skip_ref):
        deg = jnp.sum(deg_ref[...], axis=1, keepdims=True) + 1.0
        isd = lax.rsqrt(deg)
        h = jnp.dot(x_ref[...], wm_ref[...], preferred_element_type=jnp.float32)
        h2_ref[...] = (h + bm_ref[...]) * isd
        skip_ref[...] = (
            jnp.dot(x_ref[...], wl_ref[...], preferred_element_type=jnp.float32)
            + bl_ref[...])

    return pl.pallas_call(
        body,
        grid=(np_pad // block_rows,),
        in_specs=[
            pl.BlockSpec((block_rows, d), lambda i: (i, 0)),
            pl.BlockSpec((d, d), lambda i: (0, 0)),
            pl.BlockSpec((d,), lambda i: (0,)),
            pl.BlockSpec((d, d), lambda i: (0, 0)),
            pl.BlockSpec((d,), lambda i: (0,)),
            pl.BlockSpec((block_rows, NUM_WORKERS), lambda i: (i, 0)),
        ],
        out_specs=[
            pl.BlockSpec((block_rows, d), lambda i: (i, 0)),
            pl.BlockSpec((block_rows, d), lambda i: (i, 0)),
        ],
        out_shape=[
            jax.ShapeDtypeStruct((np_pad, d), jnp.float32),
            jax.ShapeDtypeStruct((np_pad, d), jnp.float32),
        ],
    )(x_pad, W_msg, b_msg, W_lin, b_lin, deg_t)


def _tc_final(agg_parts, deg_t, skip, block_rows):
    _, np_pad, d = agg_parts.shape

    def body(agg_ref, deg_ref, skip_ref, out_ref):
        deg = jnp.sum(deg_ref[...], axis=1, keepdims=True) + 1.0
        isd = lax.rsqrt(deg)
        seg = agg_ref[0] + agg_ref[1]
        out_ref[...] = jnp.maximum(seg * isd + skip_ref[...], 0.0)

    return pl.pallas_call(
        body,
        grid=(np_pad // block_rows,),
        in_specs=[
            pl.BlockSpec((NUM_CORES, block_rows, d), lambda i: (0, i, 0)),
            pl.BlockSpec((block_rows, NUM_WORKERS), lambda i: (i, 0)),
            pl.BlockSpec((block_rows, d), lambda i: (i, 0)),
        ],
        out_specs=pl.BlockSpec((block_rows, d), lambda i: (i, 0)),
        out_shape=jax.ShapeDtypeStruct((np_pad, d), jnp.float32),
    )(agg_parts, deg_t, skip)


def kernel(x, edge_index, W_msg, b_msg, W_lin, b_lin):
    n, d = x.shape
    e = edge_index.shape[1]
    np_pad = ((n + 2047) // 2048) * 2048
    e_unit = CHUNK * NUM_WORKERS * 8
    e_pad = ((e + e_unit - 1) // e_unit) * e_unit

    x_pad = jnp.pad(x, ((0, np_pad - n), (0, 0)))
    ei_pad = jnp.pad(edge_index, ((0, 0), (0, e_pad - e)), constant_values=n)
    src2d = ei_pad[0].reshape(e_pad // CHUNK, CHUNK)
    dst2d = ei_pad[1].reshape(e_pad // CHUNK, CHUNK)

    deg_parts = _sc_degree(dst2d, np_pad)
    deg_t = deg_parts.reshape(NUM_WORKERS, np_pad).T
    h2, skip = _tc_transform(x_pad, W_msg, b_msg, W_lin, b_lin, deg_t, 640)
    agg_parts = _sc_gather_scatter(h2, src2d, dst2d)
    out = _tc_final(agg_parts, deg_t, skip, 640)
    return out[:n]
